# baseline (speedup 1.0000x reference)
import functools

import jax
import jax.numpy as jnp
from jax import lax
from jax.experimental import pallas as pl
from jax.experimental.pallas import tpu as pltpu
from jax.experimental.pallas import tpu_sc as plsc

NC = 2
NS = 16
NW = NC * NS
L = 16
CHUNK = 128


def _mf_body(users_hbm, items_hbm, uf_hbm, if_hbm, ub_hbm, ib_hbm, out_hbm,
             idx_u, idx_i, uf_v, if_v, ub_v, ib_v, out_v, sem,
             *, b_per_w, factors):
  wid = lax.axis_index("s") * NC + lax.axis_index("c")
  base = wid * b_per_w
  nchunk = b_per_w // CHUNK

  pltpu.sync_copy(users_hbm.at[pl.ds(base, b_per_w)], idx_u)
  pltpu.sync_copy(items_hbm.at[pl.ds(base, b_per_w)], idx_i)

  copies = []
  for j in range(nchunk):
    s = pl.ds(j * CHUNK, CHUNK)
    copies.append(pltpu.async_copy(uf_hbm.at[idx_u.at[s]], uf_v.at[s], sem))
    copies.append(pltpu.async_copy(if_hbm.at[idx_i.at[s]], if_v.at[s], sem))
    copies.append(pltpu.async_copy(ub_hbm.at[idx_u.at[s]], ub_v.at[s], sem))
    copies.append(pltpu.async_copy(ib_hbm.at[idx_i.at[s]], ib_v.at[s], sem))
  for c in copies:
    c.wait()

  zeros = jnp.zeros((L,), jnp.int32)
  iota = lax.iota(jnp.int32, L)

  def group(g, carry):
    rows = g * L + iota
    acc = ub_v[pl.ds(g * L, L)] + ib_v[pl.ds(g * L, L)]
    for f in range(factors):
      col = jnp.full((L,), f, jnp.int32)
      acc += (plsc.load_gather(uf_v, [rows, col]) *
              plsc.load_gather(if_v, [rows, col]))
    out_v[pl.ds(g * L, L)] = acc
    return carry

  lax.fori_loop(0, b_per_w // L, group, 0)

  pltpu.sync_copy(out_v, out_hbm.at[pl.ds(base, b_per_w)])


def kernel(users, items, user_factors, item_factors, user_bias, item_bias):
  b = users.shape[0]
  factors = user_factors.shape[1]
  assert b % (NW * CHUNK) == 0
  b_per_w = b // NW

  users = users.astype(jnp.int32)
  items = items.astype(jnp.int32)

  mesh = plsc.VectorSubcoreMesh(core_axis_name="c", subcore_axis_name="s",
                                num_cores=NC, num_subcores=NS)
  body = functools.partial(_mf_body, b_per_w=b_per_w, factors=factors)
  run = pl.kernel(
      body,
      out_type=jax.ShapeDtypeStruct((b,), jnp.float32),
      mesh=mesh,
      scratch_types=[
          pltpu.VMEM((b_per_w,), jnp.int32),
          pltpu.VMEM((b_per_w,), jnp.int32),
          pltpu.VMEM((b_per_w, factors), jnp.float32),
          pltpu.VMEM((b_per_w, factors), jnp.float32),
          pltpu.VMEM((b_per_w,), jnp.float32),
          pltpu.VMEM((b_per_w,), jnp.float32),
          pltpu.VMEM((b_per_w,), jnp.float32),
          pltpu.SemaphoreType.DMA,
      ],
      compiler_params=pltpu.CompilerParams(needs_layout_passes=False,
                                           use_tc_tiling_on_sc=False),
  )
  return run(users, items, user_factors, item_factors,
             user_bias.reshape(-1), item_bias.reshape(-1))
